# baseline (speedup 1.0000x reference)
import functools

import jax
import jax.numpy as jnp
from jax import lax
from jax.experimental import pallas as pl
from jax.experimental.pallas import tpu as pltpu
from jax.experimental.pallas import tpu_sc as plsc

_NC, _NS = 2, 16
_NW = _NC * _NS


def _gather_body(bw, u_hbm, i_hbm, ub_hbm, uv_hbm, ib_hbm, iv_hbm,
                 vu_out, vi_out, bu_out, bi_out,
                 uidx_v, iidx_v, vu_v, vi_v, bu_v, bi_v, sem):
    wid = lax.axis_index("s") * _NC + lax.axis_index("c")
    base = wid * bw
    pltpu.sync_copy(u_hbm.at[pl.ds(base, bw)], uidx_v)
    pltpu.sync_copy(i_hbm.at[pl.ds(base, bw)], iidx_v)
    c1 = pltpu.async_copy(uv_hbm.at[uidx_v], vu_v, sem)
    c2 = pltpu.async_copy(iv_hbm.at[iidx_v], vi_v, sem)
    c3 = pltpu.async_copy(ub_hbm.at[uidx_v], bu_v, sem)
    c4 = pltpu.async_copy(ib_hbm.at[iidx_v], bi_v, sem)
    c1.wait()
    pltpu.sync_copy(vu_v, vu_out.at[pl.ds(base, bw)])
    c2.wait()
    pltpu.sync_copy(vi_v, vi_out.at[pl.ds(base, bw)])
    c3.wait()
    pltpu.sync_copy(bu_v, bu_out.at[pl.ds(base, bw)])
    c4.wait()
    pltpu.sync_copy(bi_v, bi_out.at[pl.ds(base, bw)])


def _sc_gather(u, i, user_bias, user_vec, item_bias, item_vec):
    B = u.shape[0]
    D = user_vec.shape[1]
    assert B % (8 * _NW) == 0
    bw = B // _NW
    mesh = plsc.VectorSubcoreMesh(core_axis_name="c", subcore_axis_name="s",
                                  num_cores=_NC, num_subcores=_NS)
    f32 = jnp.float32
    k = pl.kernel(
        functools.partial(_gather_body, bw),
        out_type=(
            jax.ShapeDtypeStruct((B, D), f32),
            jax.ShapeDtypeStruct((B, D), f32),
            jax.ShapeDtypeStruct((B,), f32),
            jax.ShapeDtypeStruct((B,), f32),
        ),
        mesh=mesh,
        scratch_types=[
            pltpu.VMEM((bw,), jnp.int32),
            pltpu.VMEM((bw,), jnp.int32),
            pltpu.VMEM((bw, D), f32),
            pltpu.VMEM((bw, D), f32),
            pltpu.VMEM((bw,), f32),
            pltpu.VMEM((bw,), f32),
            pltpu.SemaphoreType.DMA,
        ],
        compiler_params=pltpu.CompilerParams(use_tc_tiling_on_sc=False),
    )
    return k(u, i, user_bias, user_vec, item_bias, item_vec)


def _dense_body(vu_ref, vi_ref, bu_ref, bi_ref, w1t_ref, b1_ref, w2t_ref,
                b2_ref, gb_ref, out_ref):
    h1 = jnp.dot(vu_ref[...], w1t_ref[...],
                 preferred_element_type=jnp.float32) + b1_ref[...]
    h2 = jnp.dot(vi_ref[...], w2t_ref[...],
                 preferred_element_type=jnp.float32) + b2_ref[...]
    s = jnp.sum(h1 * h2, axis=1)
    out_ref[...] = s + bu_ref[...] + bi_ref[...] + gb_ref[0, 0]


def _tc_dense(vu, vi, bu, bi, W1, b1, W2, b2, glob_bias):
    B = vu.shape[0]
    return pl.pallas_call(
        _dense_body,
        out_shape=jax.ShapeDtypeStruct((B,), jnp.float32),
    )(vu, vi, bu, bi, W1.T, b1.reshape(1, -1), W2.T, b2.reshape(1, -1),
      glob_bias)


def kernel(u, i, glob_bias, user_bias, user_vec, item_bias, item_vec,
           W1, b1, W2, b2):
    vu, vi, bu, bi = _sc_gather(u, i, user_bias, user_vec, item_bias,
                                item_vec)
    return _tc_dense(vu, vi, bu, bi, W1, b1, W2, b2, glob_bias)
